# baseline (speedup 1.0000x reference)
import functools

import jax
import jax.numpy as jnp
from jax import lax
from jax.experimental import pallas as pl
from jax.experimental.pallas import tpu as pltpu

TOKENS = 8192
CHUNK = 1024
NBUF = 4


def _gate_pipelined(x_hbm, w_ref, o_ref, buf, sems):
    nchunks = TOKENS // CHUNK

    def chunk_copy(i):
        slot = i % NBUF
        return pltpu.make_async_copy(
            x_hbm.at[pl.ds(i * CHUNK, CHUNK), :],
            buf.at[slot],
            sems.at[slot])

    for s in range(NBUF - 1):
        chunk_copy(s).start()

    for i in range(nchunks):
        chunk_copy(i).wait()
        if i + NBUF - 1 < nchunks:
            chunk_copy(i + NBUF - 1).start()
        o_ref[pl.ds(i * CHUNK, CHUNK), :] = lax.dot_general(
            buf[i % NBUF], w_ref[...],
            dimension_numbers=(((1,), (1,)), ((), ())),
            preferred_element_type=jnp.float32)


@jax.jit
def kernel(x, W):
    tokens, model_dim = x.shape
    num_experts = W.shape[0]
    return pl.pallas_call(
        _gate_pipelined,
        in_specs=[
            pl.BlockSpec(memory_space=pltpu.MemorySpace.HBM),
            pl.BlockSpec((num_experts, model_dim), lambda: (0, 0)),
        ],
        out_specs=pl.BlockSpec((tokens, num_experts), lambda: (0, 0)),
        out_shape=jax.ShapeDtypeStruct((tokens, num_experts), jnp.float32),
        scratch_shapes=[
            pltpu.VMEM((NBUF, CHUNK, 2048), jnp.float32),
            pltpu.SemaphoreType.DMA((NBUF,)),
        ],
    )(x, W)
